# baseline (speedup 1.0000x reference)
import jax
import jax.numpy as jnp
from jax import lax
from jax.experimental import pallas as pl
from jax.experimental.pallas import tpu as pltpu
from jax.experimental.pallas import tpu_sc as plsc

VOCAB = 1000000
EMBED_DIM = 64
BATCH = 4096
SEQ = 200

NC = 2
NS = 16
NW = NC * NS

CHUNK = 128
TOTAL = BATCH * SEQ
CPW = TOTAL // (NW * CHUNK)
NBUF = 2
ROWW = 2 * EMBED_DIM

PR = 160
PCHUNKS = VOCAB // PR
PPW = PCHUNKS // NW
PEXTRA = PCHUNKS % NW


def _pad_body(tab_hbm, wide_hbm, in_v, out_v, isem, osem):
  wid = lax.axis_index("s") * NC + lax.axis_index("c")
  n_mine = jnp.where(wid < PEXTRA, PPW + 1, PPW)

  def body(i, buf):
    c = i * NW + wid
    @pl.when(i >= 2)
    def _():
      pltpu.make_async_copy(
          out_v.at[buf], wide_hbm.at[pl.ds(0, PR)], osem.at[buf]).wait()
    pltpu.make_async_copy(
        tab_hbm.at[pl.ds(c * PR, PR)], in_v.at[buf], isem.at[buf]).wait()

    def widen(r, carry):
      for k in range(EMBED_DIM // 16):
        sl = pl.ds(k * 16, 16)
        out_v[buf, r, sl] = in_v[buf, r, sl]
      return carry
    lax.fori_loop(0, PR, widen, 0)

    pltpu.async_copy(
        out_v.at[buf], wide_hbm.at[pl.ds(c * PR, PR)], osem.at[buf])
    @pl.when(i + 1 < n_mine)
    def _():
      pltpu.async_copy(
          tab_hbm.at[pl.ds(((i + 1) * NW + wid) * PR, PR)],
          in_v.at[1 - buf], isem.at[1 - buf])

  def pair(g, carry):
    for buf in range(2):
      i = g * 2 + buf
      @pl.when(i < n_mine)
      def _():
        body(i, buf)
    return carry

  pltpu.async_copy(tab_hbm.at[pl.ds(wid * PR, PR)], in_v.at[0], isem.at[0])
  lax.fori_loop(0, (PPW + 2) // 2, pair, 0)

  for buf in range(2):
    pltpu.make_async_copy(
        out_v.at[buf], wide_hbm.at[pl.ds(0, PR)], osem.at[buf]).wait()


def _gather_body(tab_hbm, idx_hbm, out_hbm, idx_v, rows_v, gsem):
  wid = lax.axis_index("s") * NC + lax.axis_index("c")
  row_base = wid * (CPW * CHUNK)

  pltpu.sync_copy(idx_hbm.at[pl.ds(wid * CPW, CPW)], idx_v)

  for b in range(NBUF):
    pltpu.async_copy(tab_hbm.at[idx_v.at[b]], rows_v.at[b], gsem.at[b])

  def group(g, carry):
    for b in range(NBUF):
      j = g * NBUF + b
      pltpu.make_async_copy(
          tab_hbm.at[idx_v.at[j]], rows_v.at[b], gsem.at[b]).wait()
      pltpu.sync_copy(
          rows_v.at[b], out_hbm.at[pl.ds(row_base + j * CHUNK, CHUNK)])
      jn = j + NBUF
      pltpu.async_copy(tab_hbm.at[idx_v.at[jn]], rows_v.at[b], gsem.at[b])
    return carry

  lax.fori_loop(0, CPW // NBUF - 1, group, 0)

  for b in range(NBUF):
    j = (CPW - NBUF) + b
    pltpu.make_async_copy(
        tab_hbm.at[idx_v.at[j]], rows_v.at[b], gsem.at[b]).wait()
    pltpu.sync_copy(
        rows_v.at[b], out_hbm.at[pl.ds(row_base + j * CHUNK, CHUNK)])


_cache = {}


def _mesh():
  return plsc.VectorSubcoreMesh(
      core_axis_name="c", subcore_axis_name="s",
      num_cores=NC, num_subcores=NS)


def _get_pad():
  if "pad" not in _cache:
    _cache["pad"] = pl.kernel(
        _pad_body,
        out_type=jax.ShapeDtypeStruct((VOCAB, ROWW), jnp.float32),
        mesh=_mesh(),
        scratch_types=[
            pltpu.VMEM((2, PR, EMBED_DIM), jnp.float32),
            pltpu.VMEM((2, PR, ROWW), jnp.float32),
            pltpu.SemaphoreType.DMA((2,)),
            pltpu.SemaphoreType.DMA((2,)),
        ],
        compiler_params=pltpu.CompilerParams(use_tc_tiling_on_sc=True),
    )
  return _cache["pad"]


def _get_gather():
  if "gather" not in _cache:
    _cache["gather"] = pl.kernel(
        _gather_body,
        out_type=jax.ShapeDtypeStruct((TOTAL, ROWW), jnp.float32),
        mesh=_mesh(),
        scratch_types=[
            pltpu.VMEM((CPW, CHUNK), jnp.int32),
            pltpu.VMEM((NBUF, CHUNK, ROWW), jnp.float32),
            pltpu.SemaphoreType.DMA((NBUF,)),
        ],
        compiler_params=pltpu.CompilerParams(use_tc_tiling_on_sc=True),
    )
  return _cache["gather"]


@jax.jit
def kernel(words, table):
  idx = words.reshape(TOTAL // CHUNK, CHUNK)
  wide_tab = _get_pad()(table)
  wide = _get_gather()(wide_tab, idx)
  return wide[:, :EMBED_DIM].reshape(BATCH, SEQ, EMBED_DIM)


# speedup vs baseline: 1.3058x; 1.3058x over previous
import jax
import jax.numpy as jnp
from jax import lax
from jax.experimental import pallas as pl
from jax.experimental.pallas import tpu as pltpu
from jax.experimental.pallas import tpu_sc as plsc

VOCAB = 1000000
EMBED_DIM = 64
BATCH = 4096
SEQ = 200

NC = 2
NS = 16
NW = NC * NS

CHUNK = 128
TOTAL = BATCH * SEQ
CPW = TOTAL // (NW * CHUNK)
NBUF = 5
ROWW = 2 * EMBED_DIM


def _body(tab_hbm, idx_hbm, out_hbm, idx_v, rows_v, gsem):
  wid = lax.axis_index("s") * NC + lax.axis_index("c")
  row_base = wid * (CPW * CHUNK)

  pltpu.sync_copy(idx_hbm.at[pl.ds(wid * CPW, CPW)], idx_v)

  for b in range(NBUF):
    pltpu.async_copy(tab_hbm.at[idx_v.at[b]], rows_v.at[b], gsem.at[b])

  def group(g, carry):
    for b in range(NBUF):
      j = g * NBUF + b
      pltpu.make_async_copy(
          tab_hbm.at[idx_v.at[j]], rows_v.at[b], gsem.at[b]).wait()
      pltpu.sync_copy(
          rows_v.at[b], out_hbm.at[pl.ds(row_base + j * CHUNK, CHUNK)])
      jn = j + NBUF
      pltpu.async_copy(tab_hbm.at[idx_v.at[jn]], rows_v.at[b], gsem.at[b])
    return carry

  lax.fori_loop(0, CPW // NBUF - 1, group, 0)

  for b in range(NBUF):
    j = (CPW - NBUF) + b
    pltpu.make_async_copy(
        tab_hbm.at[idx_v.at[j]], rows_v.at[b], gsem.at[b]).wait()
    pltpu.sync_copy(
        rows_v.at[b], out_hbm.at[pl.ds(row_base + j * CHUNK, CHUNK)])


_gather_cache = []


def _get_gather():
  if not _gather_cache:
    _gather_cache.append(pl.kernel(
        _body,
        out_type=jax.ShapeDtypeStruct((TOTAL, ROWW), jnp.float32),
        mesh=plsc.VectorSubcoreMesh(
            core_axis_name="c", subcore_axis_name="s",
            num_cores=NC, num_subcores=NS),
        scratch_types=[
            pltpu.VMEM((CPW, CHUNK), jnp.int32),
            pltpu.VMEM((NBUF, CHUNK, ROWW), jnp.float32),
            pltpu.SemaphoreType.DMA((NBUF,)),
        ],
        compiler_params=pltpu.CompilerParams(use_tc_tiling_on_sc=False),
    ))
  return _gather_cache[0]


@jax.jit
def kernel(words, table):
  idx = words.reshape(TOTAL // CHUNK, CHUNK)
  tab = jnp.pad(table, ((0, 0), (0, ROWW - EMBED_DIM)))
  wide = _get_gather()(tab, idx)
  return wide[:, :EMBED_DIM].reshape(BATCH, SEQ, EMBED_DIM)
